# baseline (speedup 1.0000x reference)
import jax
import jax.numpy as jnp
from jax import lax
from jax.experimental import pallas as pl
from jax.experimental.pallas import tpu as pltpu
from jax.experimental.pallas import tpu_sc as plsc

N_NODES = 100000
D = 128
NUM_WORKERS = 32
NODES_PER_TILE = 3136
LAST_TILE_NODES = N_NODES - (NUM_WORKERS - 1) * NODES_PER_TILE
BN = 64
CHUNKS_PER_TILE = NODES_PER_TILE // BN
LAST_FULL_CHUNKS = LAST_TILE_NODES // BN
LAST_PART = LAST_TILE_NODES - LAST_FULL_CHUNKS * BN
OUTER = (CHUNKS_PER_TILE + 1) // 2


def _body(e_hbm, a_hbm, c_hbm, h_hbm,
          we_hbm, wa_hbm, wc_hbm, wh_hbm,
          out_hbm,
          ei, ai, ci, hi,
          ob0, ob1,
          g0, g1, c0, c1):
    info = plsc.get_sparse_core_info()
    nc = info.num_cores
    wid = lax.axis_index("s") * nc + lax.axis_index("c")
    base_node = wid * NODES_PER_TILE
    is_last = wid == NUM_WORKERS - 1

    idx_refs = (ei, ai, ci, hi)
    src_refs = (e_hbm, a_hbm, c_hbm, h_hbm)
    tab_refs = (we_hbm, wa_hbm, wc_hbm, wh_hbm)

    @pl.when(jnp.logical_not(is_last))
    def _():
        for f in range(4):
            pltpu.sync_copy(src_refs[f].at[pl.ds(base_node, NODES_PER_TILE)],
                            idx_refs[f].at[pl.ds(0, NODES_PER_TILE)])

    @pl.when(is_last)
    def _():
        for f in range(4):
            pltpu.sync_copy(src_refs[f].at[pl.ds(base_node, LAST_TILE_NODES)],
                            idx_refs[f].at[pl.ds(0, LAST_TILE_NODES)])

    obufs = (ob0, ob1)
    gsems = (g0, g1)
    csems = (c0, c1)

    def chunk_work(chunk, k, nb):
        ob = obufs[k]
        cbase = chunk * BN
        for f in range(4):
            pltpu.async_copy(tab_refs[f].at[idx_refs[f].at[pl.ds(cbase, nb)]],
                             ob.at[pl.ds(0, nb), f], gsems[k])
        for f in range(4):
            pltpu.make_async_copy(
                tab_refs[f].at[idx_refs[f].at[pl.ds(cbase, nb)]],
                ob.at[pl.ds(0, nb), f], gsems[k]).wait()
        oofs = base_node + cbase
        pltpu.async_copy(ob.at[pl.ds(0, nb)],
                         out_hbm.at[pl.ds(oofs, nb)], csems[k])

    def outer_body(o, carry):
        for k in range(2):
            chunk = o * 2 + k
            n_chunks = jnp.where(is_last, LAST_FULL_CHUNKS + 1,
                                 CHUNKS_PER_TILE)

            @pl.when(chunk < n_chunks)
            def _():
                @pl.when(o > 0)
                def _():
                    pltpu.make_async_copy(
                        obufs[k], out_hbm.at[pl.ds(0, BN)], csems[k]).wait()

                is_part = jnp.logical_and(is_last, chunk == LAST_FULL_CHUNKS)

                @pl.when(jnp.logical_not(is_part))
                def _():
                    chunk_work(chunk, k, BN)

                @pl.when(is_part)
                def _():
                    chunk_work(chunk, k, LAST_PART)

        return carry

    lax.fori_loop(0, OUTER, outer_body, 0)

    pltpu.make_async_copy(
        obufs[0], out_hbm.at[pl.ds(0, BN)], csems[0]).wait()

    @pl.when(jnp.logical_not(is_last))
    def _():
        pltpu.make_async_copy(
            obufs[1], out_hbm.at[pl.ds(0, BN)], csems[1]).wait()

    @pl.when(is_last)
    def _():
        pltpu.make_async_copy(
            obufs[1].at[pl.ds(0, LAST_PART)],
            out_hbm.at[pl.ds(0, LAST_PART)], csems[1]).wait()


@jax.jit
def kernel(element, aromatic, charge, hcount,
           W_element, W_aromatic, W_charge, W_hcount):
    mesh = plsc.VectorSubcoreMesh(core_axis_name="c", subcore_axis_name="s")
    run = pl.kernel(
        _body,
        out_type=jax.ShapeDtypeStruct((N_NODES, 4, D), jnp.float32),
        mesh=mesh,
        compiler_params=pltpu.CompilerParams(needs_layout_passes=False),
        scratch_types=(
            [pltpu.VMEM((NODES_PER_TILE,), jnp.int32) for _ in range(4)]
            + [pltpu.VMEM((BN, 4, D), jnp.float32) for _ in range(2)]
            + [pltpu.SemaphoreType.DMA for _ in range(4)]
        ),
    )
    out = run(element.astype(jnp.int32), aromatic.astype(jnp.int32),
              charge.astype(jnp.int32), hcount.astype(jnp.int32),
              W_element, W_aromatic, W_charge, W_hcount)
    return out.reshape(N_NODES, 4 * D)


# speedup vs baseline: 7.4436x; 7.4436x over previous
import jax
import jax.numpy as jnp
from jax import lax
from jax.experimental import pallas as pl
from jax.experimental.pallas import tpu as pltpu
from jax.experimental.pallas import tpu_sc as plsc

N_NODES = 100000
D = 128
OUT_D = 512
NUM_WORKERS = 32
NODES_PER_TILE = 3136
LAST_TILE_NODES = N_NODES - (NUM_WORKERS - 1) * NODES_PER_TILE
GROUPS_PER_TILE = NODES_PER_TILE // 16
LAST_TILE_GROUPS = LAST_TILE_NODES // 16
NBUF = 4
OUTER = GROUPS_PER_TILE // NBUF
GROUP_WORDS = 16 * OUT_D

OFF_A = 100 * D
OFF_C = OFF_A + 2 * D
OFF_H = OFF_C + 9 * D
T_WORDS = OFF_H + 9 * D


def _body(e_hbm, a_hbm, c_hbm, h_hbm,
          we_hbm, wa_hbm, wc_hbm, wh_hbm,
          out_hbm,
          tab, ei, ai, ci, hi,
          o0, o1, o2, o3,
          s0, s1, s2, s3):
    info = plsc.get_sparse_core_info()
    nc = info.num_cores
    wid = lax.axis_index("s") * nc + lax.axis_index("c")
    base_node = wid * NODES_PER_TILE
    n_groups = jnp.where(wid == NUM_WORKERS - 1,
                         LAST_TILE_GROUPS, GROUPS_PER_TILE)

    pltpu.sync_copy(we_hbm, tab.at[pl.ds(0, 100 * D)])
    pltpu.sync_copy(wa_hbm, tab.at[pl.ds(OFF_A, 2 * D)])
    pltpu.sync_copy(wc_hbm, tab.at[pl.ds(OFF_C, 9 * D)])
    pltpu.sync_copy(wh_hbm, tab.at[pl.ds(OFF_H, 9 * D)])

    idx_refs = (ei, ai, ci, hi)
    src_refs = (e_hbm, a_hbm, c_hbm, h_hbm)

    @pl.when(wid < NUM_WORKERS - 1)
    def _():
        for f in range(4):
            pltpu.sync_copy(src_refs[f].at[pl.ds(base_node, NODES_PER_TILE)],
                            idx_refs[f].at[pl.ds(0, NODES_PER_TILE)])

    @pl.when(wid == NUM_WORKERS - 1)
    def _():
        for f in range(4):
            pltpu.sync_copy(src_refs[f].at[pl.ds(base_node, LAST_TILE_NODES)],
                            idx_refs[f].at[pl.ds(0, LAST_TILE_NODES)])

    obufs = (o0, o1, o2, o3)
    sems = (s0, s1, s2, s3)

    iota = lax.iota(jnp.int32, 16)
    iota_out = iota * OUT_D
    out_col_base = [iota_out + f * D for f in range(4)]

    def outer_body(o, carry):
        for k in range(NBUF):
            g = o * NBUF + k

            @pl.when(g < n_groups)
            def _():
                e = ei[pl.ds(g * 16, 16)]
                a = ai[pl.ds(g * 16, 16)]
                c = ci[pl.ds(g * 16, 16)]
                h = hi[pl.ds(g * 16, 16)]
                gbase = (e * D, a * D + OFF_A, c * D + OFF_C, h * D + OFF_H)
                ob = obufs[k]

                @pl.when(o > 0)
                def _():
                    pltpu.make_async_copy(
                        ob, out_hbm.at[pl.ds(0, GROUP_WORDS)], sems[k]).wait()

                @plsc.parallel_loop(0, D, 1, unroll=8)
                def _(t):
                    colv = (iota + t) & (D - 1)
                    for f in range(4):
                        v = plsc.load_gather(tab, [gbase[f] + colv])
                        plsc.store_scatter(ob, [out_col_base[f] + colv], v)

                oofs = (base_node + g * 16) * OUT_D
                pltpu.async_copy(
                    ob, out_hbm.at[pl.ds(oofs, GROUP_WORDS)], sems[k])

        return carry

    lax.fori_loop(0, OUTER, outer_body, 0)

    for k in range(NBUF):
        pltpu.make_async_copy(
            obufs[k], out_hbm.at[pl.ds(0, GROUP_WORDS)], sems[k]).wait()


@jax.jit
def kernel(element, aromatic, charge, hcount,
           W_element, W_aromatic, W_charge, W_hcount):
    mesh = plsc.VectorSubcoreMesh(core_axis_name="c", subcore_axis_name="s")
    run = pl.kernel(
        _body,
        out_type=jax.ShapeDtypeStruct((N_NODES * OUT_D,), jnp.float32),
        mesh=mesh,
        compiler_params=pltpu.CompilerParams(needs_layout_passes=False),
        scratch_types=(
            [pltpu.VMEM((T_WORDS,), jnp.float32)]
            + [pltpu.VMEM((NODES_PER_TILE,), jnp.int32) for _ in range(4)]
            + [pltpu.VMEM((GROUP_WORDS,), jnp.float32) for _ in range(NBUF)]
            + [pltpu.SemaphoreType.DMA for _ in range(NBUF)]
        ),
    )
    out_flat = run(element.astype(jnp.int32), aromatic.astype(jnp.int32),
                   charge.astype(jnp.int32), hcount.astype(jnp.int32),
                   W_element.reshape(-1), W_aromatic.reshape(-1),
                   W_charge.reshape(-1), W_hcount.reshape(-1))
    return out_flat.reshape(N_NODES, OUT_D)
